# Initial kernel scaffold; baseline (speedup 1.0000x reference)
import jax
import jax.numpy as jnp
from jax.experimental import pallas as pl


def kernel(x, edge_index, W1, b1, W2, b2):
    raise NotImplementedError("write your pallas kernel here")


# speedup vs baseline: 15.6387x; 15.6387x over previous
import functools

import jax
import jax.numpy as jnp
from jax import lax
from jax.experimental import pallas as pl
from jax.experimental.pallas import tpu as pltpu
from jax.experimental.pallas import tpu_sc as plsc

N = 10000
E = 320000
D = 128

NC = 2
NS = 16
NW = NC * NS

N_PAD = 10240
PAD_ROWS = N_PAD - N
CH = 128
EPW = 10112
NCH = EPW // CH
E_PAD = EPW * NW
ROWS_PT = N_PAD // NS

BLK = 512
NBLK = N_PAD // BLK

_mesh = plsc.VectorSubcoreMesh(core_axis_name="c", subcore_axis_name="s")


@functools.partial(
    pl.kernel,
    out_type=jax.ShapeDtypeStruct((NC, N_PAD, D), jnp.float32),
    mesh=_mesh,
    scratch_types=[
        pltpu.VMEM((NCH, CH), jnp.int32),
        pltpu.VMEM((NCH, CH), jnp.int32),
        pltpu.VMEM((CH, D), jnp.float32),
        pltpu.VMEM_SHARED((N_PAD, D), jnp.float32),
        pltpu.SemaphoreType.DMA,
    ],
)
def _scat_kernel(hp_hbm, src_hbm, dst_hbm, zeros_hbm, out_hbm, srcv, dstv,
                 rows_v, acc, sem):
    c = lax.axis_index("c")
    s = lax.axis_index("s")
    wid = c * NS + s

    pltpu.sync_copy(zeros_hbm, rows_v)
    for k in range(ROWS_PT // CH):
        pltpu.sync_copy(rows_v, acc.at[pl.ds(s * ROWS_PT + k * CH, CH)])
    pltpu.sync_copy(src_hbm.at[wid], srcv)
    pltpu.sync_copy(dst_hbm.at[wid], dstv)
    plsc.subcore_barrier()

    def body(j, _):
        pltpu.async_copy(hp_hbm.at[srcv.at[j]], rows_v, sem).wait()
        pltpu.sync_copy(rows_v, acc.at[dstv.at[j]], add=True)
        return 0
    lax.fori_loop(0, NCH, body, 0)

    plsc.subcore_barrier()
    pltpu.sync_copy(acc.at[pl.ds(s * ROWS_PT, ROWS_PT)],
                    out_hbm.at[c, pl.ds(s * ROWS_PT, ROWS_PT)])


def _dinv_block(deg_blk, i):
    rows = lax.broadcasted_iota(jnp.int32, (BLK, 1), 0) + i * BLK
    return jnp.where(rows < N, lax.rsqrt(deg_blk), 0.0)


def _mm1_body(deg_ref, x_ref, w_ref, out_ref):
    dinv = _dinv_block(deg_ref[...], pl.program_id(0))
    out_ref[...] = dinv * jnp.dot(x_ref[...], w_ref[...],
                                  preferred_element_type=jnp.float32)


def _mm2_body(deg_ref, s0_ref, s1_ref, hp_ref, b_ref, w_ref, out_ref):
    dinv = _dinv_block(deg_ref[...], pl.program_id(0))
    g = dinv * (s0_ref[...] + s1_ref[...] + hp_ref[...]) + b_ref[...]
    g = jnp.maximum(g, 0.0)
    out_ref[...] = dinv * jnp.dot(g, w_ref[...],
                                  preferred_element_type=jnp.float32)


def _fin_body(deg_ref, s0_ref, s1_ref, hp_ref, b_ref, out_ref):
    dinv = _dinv_block(deg_ref[...], pl.program_id(0))
    out_ref[...] = dinv * (s0_ref[...] + s1_ref[...] + hp_ref[...]) + b_ref[...]


_row_spec = pl.BlockSpec((BLK, D), lambda i: (i, 0))
_deg_spec = pl.BlockSpec((BLK, 1), lambda i: (i, 0))
_w_spec = pl.BlockSpec((D, D), lambda i: (0, 0))
_b_spec = pl.BlockSpec((1, D), lambda i: (0, 0))
_out_struct = jax.ShapeDtypeStruct((N_PAD, D), jnp.float32)


def _mm1(degv, xp, w1):
    return pl.pallas_call(
        _mm1_body, grid=(NBLK,),
        in_specs=[_deg_spec, _row_spec, _w_spec],
        out_specs=_row_spec, out_shape=_out_struct,
    )(degv, xp, w1)


def _mm2(degv, s0, s1, hp, b1, w2):
    return pl.pallas_call(
        _mm2_body, grid=(NBLK,),
        in_specs=[_deg_spec, _row_spec, _row_spec, _row_spec, _b_spec, _w_spec],
        out_specs=_row_spec, out_shape=_out_struct,
    )(degv, s0, s1, hp, b1, w2)


def _fin(degv, s0, s1, hp, b2):
    return pl.pallas_call(
        _fin_body, grid=(NBLK,),
        in_specs=[_deg_spec, _row_spec, _row_spec, _row_spec, _b_spec],
        out_specs=_row_spec, out_shape=_out_struct,
    )(degv, s0, s1, hp, b2)


def kernel(x, edge_index, W1, b1, W2, b2):
    src = edge_index[0].astype(jnp.int32)
    dst = edge_index[1].astype(jnp.int32)
    pad = N + (jnp.arange(E_PAD - E, dtype=jnp.int32) % PAD_ROWS)
    src3 = jnp.concatenate([src, pad]).reshape(NW, NCH, CH)
    dst3 = jnp.concatenate([dst, pad]).reshape(NW, NCH, CH)

    zerosd = jnp.zeros((CH, D), jnp.float32)
    onesd = jnp.ones((N_PAD, D), jnp.float32)

    cnt = _scat_kernel(onesd, src3, dst3, zerosd)
    degv = (1.0 + cnt[0, :, 0] + cnt[1, :, 0]).reshape(N_PAD, 1)

    xp = jnp.pad(x, ((0, N_PAD - N), (0, 0)))
    b1r = b1.reshape(1, D)
    b2r = b2.reshape(1, D)

    hp1 = _mm1(degv, xp, W1)
    s1 = _scat_kernel(hp1, src3, dst3, zerosd)
    hp2 = _mm2(degv, s1[0], s1[1], hp1, b1r, W2)
    s2 = _scat_kernel(hp2, src3, dst3, zerosd)
    out = _fin(degv, s2[0], s2[1], hp2, b2r)
    return out[:N]


# speedup vs baseline: 22.2901x; 1.4253x over previous
import functools

import jax
import jax.numpy as jnp
from jax import lax
from jax.experimental import pallas as pl
from jax.experimental.pallas import tpu as pltpu
from jax.experimental.pallas import tpu_sc as plsc

N = 10000
E = 320000
D = 128

NC = 2
NS = 16
NW = NC * NS

N_PAD = 10240
PAD_ROWS = N_PAD - N
CH = 128
EPW = 10240
NCH = EPW // CH
E_PAD = EPW * NW
ROWS_PT = N_PAD // NS

BLK = 512
NBLK = N_PAD // BLK

_mesh = plsc.VectorSubcoreMesh(core_axis_name="c", subcore_axis_name="s")


@functools.partial(
    pl.kernel,
    out_type=jax.ShapeDtypeStruct((NC, N_PAD, D), jnp.float32),
    mesh=_mesh,
    scratch_types=[
        pltpu.VMEM((NCH // 2, CH), jnp.int32),
        pltpu.VMEM((NCH // 2, CH), jnp.int32),
        pltpu.VMEM((CH, D), jnp.float32),
        pltpu.VMEM((CH, D), jnp.float32),
        pltpu.VMEM_SHARED((N_PAD, D), jnp.float32),
        pltpu.SemaphoreType.DMA,
        pltpu.SemaphoreType.DMA,
    ],
)
def _scat_kernel(hp_hbm, src_hbm, dst_hbm, zeros_hbm, out_hbm, srcv, dstv,
                 rows_a, rows_b, acc, sem_a, sem_b):
    c = lax.axis_index("c")
    s = lax.axis_index("s")
    wid = c * NS + s
    hch = NCH // 2

    pltpu.sync_copy(zeros_hbm, rows_a)
    for k in range(ROWS_PT // CH):
        pltpu.sync_copy(rows_a, acc.at[pl.ds(s * ROWS_PT + k * CH, CH)])
    plsc.subcore_barrier()

    for h in range(2):
        pltpu.sync_copy(src_hbm.at[wid, pl.ds(h * hch, hch)], srcv)
        pltpu.sync_copy(dst_hbm.at[wid, pl.ds(h * hch, hch)], dstv)
        pltpu.async_copy(hp_hbm.at[srcv.at[0]], rows_a, sem_a)

        def body(i, _):
            j = 2 * i
            pltpu.async_copy(hp_hbm.at[srcv.at[j + 1]], rows_b, sem_b)
            pltpu.make_async_copy(hp_hbm.at[srcv.at[j]], rows_a, sem_a).wait()
            pltpu.sync_copy(rows_a, acc.at[dstv.at[j]], add=True)
            pltpu.async_copy(hp_hbm.at[srcv.at[jnp.minimum(j + 2, hch - 1)]],
                             rows_a, sem_a)
            pltpu.make_async_copy(hp_hbm.at[srcv.at[j + 1]], rows_b,
                                  sem_b).wait()
            pltpu.sync_copy(rows_b, acc.at[dstv.at[j + 1]], add=True)
            return 0
        lax.fori_loop(0, hch // 2, body, 0)
        pltpu.make_async_copy(hp_hbm.at[srcv.at[hch - 1]], rows_a,
                              sem_a).wait()

    plsc.subcore_barrier()
    pltpu.sync_copy(acc.at[pl.ds(s * ROWS_PT, ROWS_PT)],
                    out_hbm.at[c, pl.ds(s * ROWS_PT, ROWS_PT)])


def _dinv_block(deg_blk, i):
    rows = lax.broadcasted_iota(jnp.int32, (BLK, 1), 0) + i * BLK
    return jnp.where(rows < N, lax.rsqrt(deg_blk), 0.0)


def _mm1_body(deg_ref, x_ref, w_ref, out_ref):
    dinv = _dinv_block(deg_ref[...], pl.program_id(0))
    out_ref[...] = dinv * jnp.dot(x_ref[...], w_ref[...],
                                  preferred_element_type=jnp.float32)


def _mm2_body(deg_ref, s0_ref, s1_ref, hp_ref, b_ref, w_ref, out_ref):
    dinv = _dinv_block(deg_ref[...], pl.program_id(0))
    g = dinv * (s0_ref[...] + s1_ref[...] + hp_ref[...]) + b_ref[...]
    g = jnp.maximum(g, 0.0)
    out_ref[...] = dinv * jnp.dot(g, w_ref[...],
                                  preferred_element_type=jnp.float32)


def _fin_body(deg_ref, s0_ref, s1_ref, hp_ref, b_ref, out_ref):
    dinv = _dinv_block(deg_ref[...], pl.program_id(0))
    out_ref[...] = dinv * (s0_ref[...] + s1_ref[...] + hp_ref[...]) + b_ref[...]


_row_spec = pl.BlockSpec((BLK, D), lambda i: (i, 0))
_deg_spec = pl.BlockSpec((BLK, 1), lambda i: (i, 0))
_w_spec = pl.BlockSpec((D, D), lambda i: (0, 0))
_b_spec = pl.BlockSpec((1, D), lambda i: (0, 0))
_out_struct = jax.ShapeDtypeStruct((N_PAD, D), jnp.float32)


def _mm1(degv, xp, w1):
    return pl.pallas_call(
        _mm1_body, grid=(NBLK,),
        in_specs=[_deg_spec, _row_spec, _w_spec],
        out_specs=_row_spec, out_shape=_out_struct,
    )(degv, xp, w1)


def _mm2(degv, s0, s1, hp, b1, w2):
    return pl.pallas_call(
        _mm2_body, grid=(NBLK,),
        in_specs=[_deg_spec, _row_spec, _row_spec, _row_spec, _b_spec, _w_spec],
        out_specs=_row_spec, out_shape=_out_struct,
    )(degv, s0, s1, hp, b1, w2)


def _fin(degv, s0, s1, hp, b2):
    return pl.pallas_call(
        _fin_body, grid=(NBLK,),
        in_specs=[_deg_spec, _row_spec, _row_spec, _row_spec, _b_spec],
        out_specs=_row_spec, out_shape=_out_struct,
    )(degv, s0, s1, hp, b2)


def kernel(x, edge_index, W1, b1, W2, b2):
    src = edge_index[0].astype(jnp.int32)
    dst = edge_index[1].astype(jnp.int32)
    pad = N + (jnp.arange(E_PAD - E, dtype=jnp.int32) % PAD_ROWS)
    src3 = jnp.concatenate([src, pad]).reshape(NW, NCH, CH)
    dst3 = jnp.concatenate([dst, pad]).reshape(NW, NCH, CH)

    zerosd = jnp.zeros((CH, D), jnp.float32)
    onesd = jnp.ones((N_PAD, D), jnp.float32)

    cnt = _scat_kernel(onesd, src3, dst3, zerosd)
    degv = (1.0 + cnt[0, :, 0] + cnt[1, :, 0]).reshape(N_PAD, 1)

    xp = jnp.pad(x, ((0, N_PAD - N), (0, 0)))
    b1r = b1.reshape(1, D)
    b2r = b2.reshape(1, D)

    hp1 = _mm1(degv, xp, W1)
    s1 = _scat_kernel(hp1, src3, dst3, zerosd)
    hp2 = _mm2(degv, s1[0], s1[1], hp1, b1r, W2)
    s2 = _scat_kernel(hp2, src3, dst3, zerosd)
    out = _fin(degv, s2[0], s2[1], hp2, b2r)
    return out[:N]


# speedup vs baseline: 24.7032x; 1.1083x over previous
import functools

import jax
import jax.numpy as jnp
from jax import lax
from jax.experimental import pallas as pl
from jax.experimental.pallas import tpu as pltpu
from jax.experimental.pallas import tpu_sc as plsc

N = 10000
E = 320000
D = 128

NC = 2
NS = 16
NW = NC * NS

N_PAD = 10240
PAD_ROWS = N_PAD - N
CH = 128
EPW = 10240
NCH = EPW // CH
E_PAD = EPW * NW
ROWS_PT = N_PAD // NS

BLK = 512
NBLK = N_PAD // BLK

_mesh = plsc.VectorSubcoreMesh(core_axis_name="c", subcore_axis_name="s")


@functools.partial(
    pl.kernel,
    out_type=jax.ShapeDtypeStruct((NC, N_PAD, D), jnp.float32),
    mesh=_mesh,
    scratch_types=[
        pltpu.VMEM((NCH, CH), jnp.int32),
        pltpu.VMEM((CH, D), jnp.float32),
        pltpu.VMEM_SHARED((N_PAD, D), jnp.float32),
        pltpu.SemaphoreType.DMA,
    ],
)
def _deg_kernel(dst_hbm, out_hbm, dstv, ones_v, acc, sem):
    c = lax.axis_index("c")
    s = lax.axis_index("s")
    wid = c * NS + s

    def fill(val):
        def row(r, _):
            for k in range(D // 16):
                ones_v[r, pl.ds(k * 16, 16)] = jnp.full((16,), val,
                                                        jnp.float32)
            return 0
        lax.fori_loop(0, CH, row, 0)

    fill(0.0)
    for k in range(ROWS_PT // CH):
        pltpu.sync_copy(ones_v, acc.at[pl.ds(s * ROWS_PT + k * CH, CH)])
    fill(1.0)
    pltpu.sync_copy(dst_hbm.at[wid], dstv)
    plsc.subcore_barrier()

    pltpu.async_copy(ones_v, acc.at[dstv.at[0]], sem, add=True)
    pltpu.async_copy(ones_v, acc.at[dstv.at[1]], sem, add=True)

    def body(j, _):
        pltpu.async_copy(ones_v, acc.at[dstv.at[j + 2]], sem, add=True)
        pltpu.make_async_copy(ones_v, acc.at[dstv.at[j]], sem).wait()
        return 0
    lax.fori_loop(0, NCH - 2, body, 0)
    pltpu.make_async_copy(ones_v, acc.at[dstv.at[NCH - 2]], sem).wait()
    pltpu.make_async_copy(ones_v, acc.at[dstv.at[NCH - 1]], sem).wait()

    plsc.subcore_barrier()
    pltpu.sync_copy(acc.at[pl.ds(s * ROWS_PT, ROWS_PT)],
                    out_hbm.at[c, pl.ds(s * ROWS_PT, ROWS_PT)])


@functools.partial(
    pl.kernel,
    out_type=jax.ShapeDtypeStruct((NC, N_PAD, D), jnp.float32),
    mesh=_mesh,
    scratch_types=[
        pltpu.VMEM((NCH // 2, CH), jnp.int32),
        pltpu.VMEM((NCH // 2, CH), jnp.int32),
        pltpu.VMEM((CH, D), jnp.float32),
        pltpu.VMEM((CH, D), jnp.float32),
        pltpu.VMEM_SHARED((N_PAD, D), jnp.float32),
        pltpu.SemaphoreType.DMA,
        pltpu.SemaphoreType.DMA,
    ],
)
def _scat_kernel(hp_hbm, src_hbm, dst_hbm, zeros_hbm, out_hbm, srcv, dstv,
                 rows_a, rows_b, acc, sem_a, sem_b):
    c = lax.axis_index("c")
    s = lax.axis_index("s")
    wid = c * NS + s
    hch = NCH // 2

    pltpu.sync_copy(zeros_hbm, rows_a)
    for k in range(ROWS_PT // CH):
        pltpu.sync_copy(rows_a, acc.at[pl.ds(s * ROWS_PT + k * CH, CH)])
    plsc.subcore_barrier()

    for h in range(2):
        pltpu.sync_copy(src_hbm.at[wid, pl.ds(h * hch, hch)], srcv)
        pltpu.sync_copy(dst_hbm.at[wid, pl.ds(h * hch, hch)], dstv)
        pltpu.async_copy(hp_hbm.at[srcv.at[0]], rows_a, sem_a)

        def body(i, _):
            j = 2 * i
            pltpu.async_copy(hp_hbm.at[srcv.at[j + 1]], rows_b, sem_b)
            pltpu.make_async_copy(hp_hbm.at[srcv.at[j]], rows_a, sem_a).wait()
            pltpu.sync_copy(rows_a, acc.at[dstv.at[j]], add=True)
            pltpu.async_copy(hp_hbm.at[srcv.at[jnp.minimum(j + 2, hch - 1)]],
                             rows_a, sem_a)
            pltpu.make_async_copy(hp_hbm.at[srcv.at[j + 1]], rows_b,
                                  sem_b).wait()
            pltpu.sync_copy(rows_b, acc.at[dstv.at[j + 1]], add=True)
            return 0
        lax.fori_loop(0, hch // 2, body, 0)
        pltpu.make_async_copy(hp_hbm.at[srcv.at[hch - 1]], rows_a,
                              sem_a).wait()

    plsc.subcore_barrier()
    pltpu.sync_copy(acc.at[pl.ds(s * ROWS_PT, ROWS_PT)],
                    out_hbm.at[c, pl.ds(s * ROWS_PT, ROWS_PT)])


def _mm1_body(c0_ref, c1_ref, x_ref, w_ref, out_ref, dv_ref):
    rows = (lax.broadcasted_iota(jnp.int32, (BLK, 1), 0)
            + pl.program_id(0) * BLK)
    deg = 1.0 + c0_ref[:, 0:1] + c1_ref[:, 0:1]
    dinv = jnp.where(rows < N, lax.rsqrt(deg), 0.0)
    dv_ref[...] = dinv
    out_ref[...] = dinv * jnp.dot(x_ref[...], w_ref[...],
                                  preferred_element_type=jnp.float32)


def _mm2_body(dv_ref, s0_ref, s1_ref, hp_ref, b_ref, w_ref, out_ref):
    dinv = dv_ref[...]
    g = dinv * (s0_ref[...] + s1_ref[...] + hp_ref[...]) + b_ref[...]
    g = jnp.maximum(g, 0.0)
    out_ref[...] = dinv * jnp.dot(g, w_ref[...],
                                  preferred_element_type=jnp.float32)


def _fin_body(dv_ref, s0_ref, s1_ref, hp_ref, b_ref, out_ref):
    out_ref[...] = (dv_ref[...] * (s0_ref[...] + s1_ref[...] + hp_ref[...])
                    + b_ref[...])


_row_spec = pl.BlockSpec((BLK, D), lambda i: (i, 0))
_deg_spec = pl.BlockSpec((BLK, 1), lambda i: (i, 0))
_w_spec = pl.BlockSpec((D, D), lambda i: (0, 0))
_b_spec = pl.BlockSpec((1, D), lambda i: (0, 0))
_out_struct = jax.ShapeDtypeStruct((N_PAD, D), jnp.float32)


def _mm1(c0, c1, xp, w1):
    return pl.pallas_call(
        _mm1_body, grid=(NBLK,),
        in_specs=[_row_spec, _row_spec, _row_spec, _w_spec],
        out_specs=[_row_spec, _deg_spec],
        out_shape=[_out_struct, jax.ShapeDtypeStruct((N_PAD, 1), jnp.float32)],
    )(c0, c1, xp, w1)


def _mm2(dinv, s0, s1, hp, b1, w2):
    return pl.pallas_call(
        _mm2_body, grid=(NBLK,),
        in_specs=[_deg_spec, _row_spec, _row_spec, _row_spec, _b_spec, _w_spec],
        out_specs=_row_spec, out_shape=_out_struct,
    )(dinv, s0, s1, hp, b1, w2)


def _fin(dinv, s0, s1, hp, b2):
    return pl.pallas_call(
        _fin_body, grid=(NBLK,),
        in_specs=[_deg_spec, _row_spec, _row_spec, _row_spec, _b_spec],
        out_specs=_row_spec, out_shape=_out_struct,
    )(dinv, s0, s1, hp, b2)


def kernel(x, edge_index, W1, b1, W2, b2):
    src = edge_index[0].astype(jnp.int32)
    dst = edge_index[1].astype(jnp.int32)
    pad = N + (jnp.arange(E_PAD - E, dtype=jnp.int32) % PAD_ROWS)
    src3 = jnp.concatenate([src, pad]).reshape(NW, NCH, CH)
    dst3 = jnp.concatenate([dst, pad]).reshape(NW, NCH, CH)

    zerosd = jnp.zeros((CH, D), jnp.float32)

    cnt = _deg_kernel(dst3)

    xp = jnp.pad(x, ((0, N_PAD - N), (0, 0)))
    b1r = b1.reshape(1, D)
    b2r = b2.reshape(1, D)

    hp1, dinv = _mm1(cnt[0], cnt[1], xp, W1)
    s1 = _scat_kernel(hp1, src3, dst3, zerosd)
    hp2 = _mm2(dinv, s1[0], s1[1], hp1, b1r, W2)
    s2 = _scat_kernel(hp2, src3, dst3, zerosd)
    out = _fin(dinv, s2[0], s2[1], hp2, b2r)
    return out[:N]
